# Initial kernel scaffold; baseline (speedup 1.0000x reference)
import jax
import jax.numpy as jnp
from jax.experimental import pallas as pl


def kernel(x, rbf, sbf, idx_kj, idx_ji, params):
    raise NotImplementedError("write your pallas kernel here")


# speedup vs baseline: 4.6637x; 4.6637x over previous
import functools

import jax
import jax.numpy as jnp
from jax import lax
from jax.experimental import pallas as pl
from jax.experimental.pallas import tpu as pltpu
from jax.experimental.pallas import tpu_sc as plsc

E = 160000
T = 640000
H = 256
I_DIM = 64
NR = 6
SBF_DIM = 42

F32 = jnp.float32
I32 = jnp.int32


def _silu(v):
    return v * jax.nn.sigmoid(v)


_BE = 1600


def _pre_body(x_ref, rbf_ref, wji, bji, wkj, bkj, wr1, wr2, wdown, xji_out, tbl_out):
    xb = x_ref[...]
    x_ji = _silu(jnp.dot(xb, wji[...], preferred_element_type=F32) + bji[...])
    x_kj = _silu(jnp.dot(xb, wkj[...], preferred_element_type=F32) + bkj[...])
    r = jnp.dot(jnp.dot(rbf_ref[...], wr1[...], preferred_element_type=F32),
                wr2[...], preferred_element_type=F32)
    tbl = _silu(jnp.dot(x_kj * r, wdown[...], preferred_element_type=F32))
    xji_out[...] = x_ji
    tbl_out[...] = tbl


def _pre(x, rbf, p):
    full = lambda shape: pl.BlockSpec(shape, lambda i: (0, 0))
    return pl.pallas_call(
        _pre_body,
        grid=(E // _BE,),
        in_specs=[
            pl.BlockSpec((_BE, H), lambda i: (i, 0)),
            pl.BlockSpec((_BE, NR), lambda i: (i, 0)),
            full((H, H)), full((1, H)),
            full((H, H)), full((1, H)),
            full((NR, 8)), full((8, H)),
            full((H, I_DIM)),
        ],
        out_specs=[
            pl.BlockSpec((_BE, H), lambda i: (i, 0)),
            pl.BlockSpec((_BE, I_DIM), lambda i: (i, 0)),
        ],
        out_shape=[
            jax.ShapeDtypeStruct((E, H), F32),
            jax.ShapeDtypeStruct((E, I_DIM), F32),
        ],
    )(x, rbf, p['W_ji'], p['b_ji'].reshape(1, H), p['W_kj'], p['b_kj'].reshape(1, H),
      p['W_rbf1'], p['W_rbf2'], p['W_down'])


_BT = 2000


def _sfeat_body(sbf_ref, w1, w2, out_ref):
    out_ref[...] = jnp.dot(jnp.dot(sbf_ref[...], w1[...], preferred_element_type=F32),
                           w2[...], preferred_element_type=F32)


def _sfeat(sbf, p):
    return pl.pallas_call(
        _sfeat_body,
        grid=(T // _BT,),
        in_specs=[
            pl.BlockSpec((_BT, SBF_DIM), lambda i: (i, 0)),
            pl.BlockSpec((SBF_DIM, 8), lambda i: (0, 0)),
            pl.BlockSpec((8, I_DIM), lambda i: (0, 0)),
        ],
        out_specs=pl.BlockSpec((_BT, I_DIM), lambda i: (i, 0)),
        out_shape=jax.ShapeDtypeStruct((T, I_DIM), F32),
    )(sbf, p['W_sbf1'], p['W_sbf2'])


_NC = 2
_NS = 16
_NW = _NC * _NS
_TPW = T // _NW
_K = 160
_NB = _TPW // _K

_CR = 20480
_TRASH = 512
_CRP = _CR + _TRASH
_NCH = 4
_EP = _NC * _NCH * _CR


def _sc_mul_body(tbl_hbm, s_hbm, kj_hbm, m_hbm, kjbuf, trows, srows, sem_t):
    cid = lax.axis_index("c")
    sid = lax.axis_index("s")
    wid = cid * _NS + sid
    base = wid * _TPW

    def batch_body(b, carry):
        tb = base + b * _K
        pltpu.sync_copy(kj_hbm.at[pl.ds(tb, _K)], kjbuf)
        cp = pltpu.async_copy(tbl_hbm.at[kjbuf], trows, sem_t)
        pltpu.sync_copy(s_hbm.at[pl.ds(tb, _K)], srows)
        cp.wait()

        def mul_body(r2, carry2):
            for c4 in range(I_DIM // 16):
                sl = pl.ds(c4 * 16, 16)
                trows[r2, sl] = trows[r2, sl] * srows[r2, sl]
            return carry2

        lax.fori_loop(0, _K, mul_body, 0)
        pltpu.sync_copy(trows, m_hbm.at[pl.ds(tb, _K)])
        return carry

    lax.fori_loop(0, _NB, batch_body, 0)


def _sc_mul(tbl, s, idx_kj):
    mesh = plsc.VectorSubcoreMesh(core_axis_name="c", subcore_axis_name="s")
    kern = functools.partial(
        pl.kernel,
        mesh=mesh,
        out_type=jax.ShapeDtypeStruct((T, I_DIM), F32),
        compiler_params=pltpu.CompilerParams(use_tc_tiling_on_sc=False),
        scratch_types=[
            pltpu.VMEM((_K,), I32),
            pltpu.VMEM((_K, I_DIM), F32),
            pltpu.VMEM((_K, I_DIM), F32),
            pltpu.SemaphoreType.DMA,
        ],
    )(_sc_mul_body)
    return kern(tbl, s, idx_kj)


_ZR = 328


def _sc_agg_body(m_hbm, ji_hbm, agg_hbm, jibuf, dstbuf, mrows, zbuf, chunk):
    cid = lax.axis_index("c")
    sid = lax.axis_index("s")
    wid = cid * _NS + sid
    base = wid * _TPW
    zeros16 = jnp.zeros((16,), F32)

    def zb_body(r2, carry):
        for c4 in range(I_DIM // 16):
            zbuf[r2, pl.ds(c4 * 16, 16)] = zeros16
        return carry

    lax.fori_loop(0, _ZR, zb_body, 0)

    rows_zero = _CRP // _NS
    rows_out = _CR // _NS

    def pass_body(pch, carry):
        lo = (cid * _NCH + pch) * _CR

        for z in range(rows_zero // _ZR):
            pltpu.sync_copy(zbuf, chunk.at[pl.ds(sid * rows_zero + z * _ZR, _ZR)])
        plsc.subcore_barrier()

        def batch_body(b, carry2):
            tb = base + b * _K
            pltpu.sync_copy(m_hbm.at[pl.ds(tb, _K)], mrows)
            pltpu.sync_copy(ji_hbm.at[pl.ds(tb, _K)], jibuf)
            for g in range(_K // 16):
                ji = jibuf[pl.ds(g * 16, 16)]
                rel = ji - lo
                dst = jnp.where((rel >= 0) & (rel < _CR), rel, _CR + (ji & (_TRASH - 1)))
                dstbuf[pl.ds(g * 16, 16)] = dst
            pltpu.sync_copy(mrows, chunk.at[dstbuf], add=True)
            return carry2

        lax.fori_loop(0, _NB, batch_body, 0)
        plsc.subcore_barrier()

        pltpu.sync_copy(chunk.at[pl.ds(sid * rows_out, rows_out)],
                        agg_hbm.at[pl.ds(lo + sid * rows_out, rows_out)])
        plsc.subcore_barrier()
        return carry

    lax.fori_loop(0, _NCH, pass_body, 0)


def _sc_agg(m, idx_ji):
    mesh = plsc.VectorSubcoreMesh(core_axis_name="c", subcore_axis_name="s")
    kern = functools.partial(
        pl.kernel,
        mesh=mesh,
        out_type=jax.ShapeDtypeStruct((_EP, I_DIM), F32),
        compiler_params=pltpu.CompilerParams(use_tc_tiling_on_sc=False),
        scratch_types=[
            pltpu.VMEM((_K,), I32),
            pltpu.VMEM((_K,), I32),
            pltpu.VMEM((_K, I_DIM), F32),
            pltpu.VMEM((_ZR, I_DIM), F32),
            pltpu.VMEM_SHARED((_CRP, I_DIM), F32),
        ],
    )(_sc_agg_body)
    return kern(m, idx_ji)


def _post_body(agg_ref, xji_ref, x_ref, wup, wbs1, bbs1, wbs2, bbs2,
               wlin, blin, wa1a, ba1a, wa1b, ba1b, wa2a, ba2a, wa2b, ba2b,
               out_ref):
    def mm(a, w, b=None):
        r = jnp.dot(a, w[...], preferred_element_type=F32)
        if b is not None:
            r = r + b[...]
        return r

    h = xji_ref[...] + _silu(mm(agg_ref[...], wup))
    h = h + _silu(mm(_silu(mm(h, wbs1, bbs1)), wbs2, bbs2))
    h = _silu(mm(h, wlin, blin)) + x_ref[...]
    h = h + _silu(mm(_silu(mm(h, wa1a, ba1a)), wa1b, ba1b))
    h = h + _silu(mm(_silu(mm(h, wa2a, ba2a)), wa2b, ba2b))
    out_ref[...] = h


def _post(agg, xji, x, p):
    full = lambda shape: pl.BlockSpec(shape, lambda i: (0, 0))
    wspecs = []
    wvals = []
    for wn, bn in (('W_bs1', 'b_bs1'), ('W_bs2', 'b_bs2'), ('W_lin', 'b_lin'),
                   ('W_as1a', 'b_as1a'), ('W_as1b', 'b_as1b'),
                   ('W_as2a', 'b_as2a'), ('W_as2b', 'b_as2b')):
        wspecs += [full((H, H)), full((1, H))]
        wvals += [p[wn], p[bn].reshape(1, H)]
    return pl.pallas_call(
        _post_body,
        grid=(E // _BE,),
        in_specs=[
            pl.BlockSpec((_BE, I_DIM), lambda i: (i, 0)),
            pl.BlockSpec((_BE, H), lambda i: (i, 0)),
            pl.BlockSpec((_BE, H), lambda i: (i, 0)),
            full((I_DIM, H)),
        ] + wspecs,
        out_specs=pl.BlockSpec((_BE, H), lambda i: (i, 0)),
        out_shape=jax.ShapeDtypeStruct((E, H), F32),
    )(agg, xji, x, p['W_up'], *wvals)


def kernel(x, rbf, sbf, idx_kj, idx_ji, params):
    xji, tbl = _pre(x, rbf, params)
    s = _sfeat(sbf, params)
    m = _sc_mul(tbl, s, idx_kj.astype(I32))
    agg = _sc_agg(m, idx_ji.astype(I32))[:E]
    return _post(agg, xji, x, params)
